# baseline (speedup 1.0000x reference)
import jax
import jax.numpy as jnp
from jax.experimental import pallas as pl
from jax.experimental.pallas import tpu as pltpu
from jax.experimental.pallas import tpu_sc as plsc

VOCAB = 100000
EMB = 64
BATCH = 1024

GATHER_WINDOW = 128
TILE = 2048


def _sc_gather_pairs(table_pairs, pair_idx):
    indices = pair_idx.reshape(1, BATCH)
    mesh = plsc.VectorSubcoreMesh(core_axis_name="core",
                                  subcore_axis_name="subcore")

    @pl.kernel(
        out_type=jax.ShapeDtypeStruct((BATCH, 2 * EMB), table_pairs.dtype),
        mesh=mesh,
    )
    def gather_kernel(table_hbm, idx_hbm, out_hbm):
        def body(idx_vmem, out_vmem):
            pltpu.sync_copy(table_hbm.at[idx_vmem.at[0]], out_vmem)

        pltpu.emit_pipeline(
            body,
            grid=(BATCH // GATHER_WINDOW,),
            in_specs=[pl.BlockSpec((1, GATHER_WINDOW),
                                   index_map=lambda i: (0, i))],
            out_specs=[pl.BlockSpec((GATHER_WINDOW, 2 * EMB),
                                    index_map=lambda i: (i, 0))],
            core_axis_name=("core", "subcore"),
            dimension_semantics=(pltpu.PARALLEL,),
        )(idx_hbm, out_hbm)

    return gather_kernel(table_pairs, indices)


def _mm_body(paired_ref, par_ref, w_ref, b_ref, out_ref):
    paired = paired_ref[...]
    par = par_ref[...]
    lat = jnp.where(par == 1, paired[:, EMB:], paired[:, :EMB])
    acc = jax.lax.dot_general(
        lat.astype(jnp.bfloat16), w_ref[...].astype(jnp.bfloat16),
        dimension_numbers=(((1,), (1,)), ((), ())),
        preferred_element_type=jnp.float32,
    )
    out_ref[...] = acc + b_ref[...]


def _tc_matmul(paired, parity, W, b):
    num_tiles = pl.cdiv(VOCAB, TILE)
    b2d = b.reshape(1, VOCAB)
    return pl.pallas_call(
        _mm_body,
        grid=(num_tiles,),
        in_specs=[
            pl.BlockSpec((BATCH, 2 * EMB), lambda j: (0, 0)),
            pl.BlockSpec((BATCH, 1), lambda j: (0, 0)),
            pl.BlockSpec((TILE, EMB), lambda j: (j, 0)),
            pl.BlockSpec((1, TILE), lambda j: (0, j)),
        ],
        out_specs=pl.BlockSpec((BATCH, TILE), lambda j: (0, j)),
        out_shape=jax.ShapeDtypeStruct((BATCH, VOCAB), jnp.float32),
        compiler_params=pltpu.CompilerParams(
            dimension_semantics=("arbitrary",),
        ),
    )(paired, parity, W, b2d)


def kernel(context, emb_table, W, b):
    table_pairs = emb_table.reshape(VOCAB // 2, 2 * EMB)
    paired = _sc_gather_pairs(table_pairs, context // 2)
    parity = (context & 1).reshape(BATCH, 1)
    return _tc_matmul(paired, parity, W, b)


# speedup vs baseline: 1.0019x; 1.0019x over previous
import jax
import jax.numpy as jnp
from jax.experimental import pallas as pl
from jax.experimental.pallas import tpu as pltpu
from jax.experimental.pallas import tpu_sc as plsc

VOCAB = 100000
EMB = 64
BATCH = 1024

GATHER_WINDOW = 128

TILE = 2048
NBUF = 4
NSTEPS = pl.cdiv(VOCAB, TILE)
TAIL = VOCAB - (NSTEPS - 1) * TILE


def _sc_gather_pairs(table_pairs, pair_idx):
    indices = pair_idx.reshape(1, BATCH)
    mesh = plsc.VectorSubcoreMesh(core_axis_name="core",
                                  subcore_axis_name="subcore")

    @pl.kernel(
        out_type=jax.ShapeDtypeStruct((BATCH, 2 * EMB), table_pairs.dtype),
        mesh=mesh,
    )
    def gather_kernel(table_hbm, idx_hbm, out_hbm):
        def body(idx_vmem, out_vmem):
            pltpu.sync_copy(table_hbm.at[idx_vmem.at[0]], out_vmem)

        pltpu.emit_pipeline(
            body,
            grid=(BATCH // GATHER_WINDOW,),
            in_specs=[pl.BlockSpec((1, GATHER_WINDOW),
                                   index_map=lambda i: (0, i))],
            out_specs=[pl.BlockSpec((GATHER_WINDOW, 2 * EMB),
                                    index_map=lambda i: (i, 0))],
            core_axis_name=("core", "subcore"),
            dimension_semantics=(pltpu.PARALLEL,),
        )(idx_hbm, out_hbm)

    return gather_kernel(table_pairs, indices)


def _tile_copy(buf, out_hbm, sems, t, k):
    return pltpu.make_async_copy(
        buf, out_hbm.at[:, pl.ds(t * TILE, TILE)], sems.at[k])


def _mm_body(paired_ref, par_ref, w_ref, b_ref, out_hbm, lat_ref,
             b0, b1, b2, b3, tail_buf, sems, tail_sem):
    j = pl.program_id(0)
    bufs = (b0, b1, b2, b3)

    @pl.when(j == 0)
    def _():
        paired = paired_ref[...]
        par = par_ref[...]
        lat = jnp.where(par == 1, paired[:, EMB:], paired[:, :EMB])
        lat_ref[...] = lat.astype(jnp.bfloat16)

    acc = jax.lax.dot_general(
        lat_ref[...], w_ref[...].astype(jnp.bfloat16),
        dimension_numbers=(((1,), (1,)), ((), ())),
        preferred_element_type=jnp.float32,
    )
    res = acc + b_ref[...]

    for k in range(NBUF):
        @pl.when((jax.lax.rem(j, NBUF) == k) & (j < NSTEPS - 1))
        def _(k=k):
            buf = bufs[k]
            @pl.when(j >= NBUF)
            def _():
                _tile_copy(buf, out_hbm, sems, j - NBUF, k).wait()
            buf[...] = res
            _tile_copy(buf, out_hbm, sems, j, k).start()

    @pl.when(j == NSTEPS - 1)
    def _():
        t_last = NSTEPS - 1
        tail_buf[...] = res[:, :TAIL]
        tail_copy = pltpu.make_async_copy(
            tail_buf, out_hbm.at[:, pl.ds(t_last * TILE, TAIL)], tail_sem)
        tail_copy.start()
        for t in range(t_last - NBUF, t_last):
            if t >= 0:
                _tile_copy(bufs[t % NBUF], out_hbm, sems, t, t % NBUF).wait()
        tail_copy.wait()


def _tc_matmul(paired, parity, W, b):
    b2d = b.reshape(1, VOCAB)
    return pl.pallas_call(
        _mm_body,
        grid=(NSTEPS,),
        in_specs=[
            pl.BlockSpec((BATCH, 2 * EMB), lambda j: (0, 0)),
            pl.BlockSpec((BATCH, 1), lambda j: (0, 0)),
            pl.BlockSpec((TILE, EMB), lambda j: (j, 0)),
            pl.BlockSpec((1, TILE), lambda j: (0, j)),
        ],
        out_specs=pl.BlockSpec(memory_space=pltpu.MemorySpace.HBM),
        out_shape=jax.ShapeDtypeStruct((BATCH, VOCAB), jnp.float32),
        scratch_shapes=[
            pltpu.VMEM((BATCH, EMB), jnp.bfloat16),
            pltpu.VMEM((BATCH, TILE), jnp.float32),
            pltpu.VMEM((BATCH, TILE), jnp.float32),
            pltpu.VMEM((BATCH, TILE), jnp.float32),
            pltpu.VMEM((BATCH, TILE), jnp.float32),
            pltpu.VMEM((BATCH, TAIL), jnp.float32),
            pltpu.SemaphoreType.DMA((NBUF,)),
            pltpu.SemaphoreType.DMA,
        ],
        compiler_params=pltpu.CompilerParams(
            dimension_semantics=("arbitrary",),
        ),
    )(paired, parity, W, b2d)


def kernel(context, emb_table, W, b):
    table_pairs = emb_table.reshape(VOCAB // 2, 2 * EMB)
    paired = _sc_gather_pairs(table_pairs, context // 2)
    parity = (context & 1).reshape(BATCH, 1)
    return _tc_matmul(paired, parity, W, b)


# speedup vs baseline: 1.1908x; 1.1885x over previous
import jax
import jax.numpy as jnp
from jax.experimental import pallas as pl
from jax.experimental.pallas import tpu as pltpu
from jax.experimental.pallas import tpu_sc as plsc

VOCAB = 100000
EMB = 64
BATCH = 1024

GATHER_WINDOW = 128

TILE = 2048
NBUF = 4
NSTEPS = pl.cdiv(VOCAB, TILE)
TAIL = VOCAB - (NSTEPS - 1) * TILE


def _sc_gather_pairs(table_pairs, pair_idx):
    indices = pair_idx.reshape(1, BATCH)
    mesh = plsc.VectorSubcoreMesh(core_axis_name="core",
                                  subcore_axis_name="subcore")

    @pl.kernel(
        out_type=jax.ShapeDtypeStruct((BATCH, 2 * EMB), table_pairs.dtype),
        mesh=mesh,
    )
    def gather_kernel(table_hbm, idx_hbm, out_hbm):
        def body(idx_vmem, out_vmem):
            pltpu.sync_copy(table_hbm.at[idx_vmem.at[0]], out_vmem)

        pltpu.emit_pipeline(
            body,
            grid=(BATCH // GATHER_WINDOW,),
            in_specs=[pl.BlockSpec((1, GATHER_WINDOW),
                                   index_map=lambda i: (0, i))],
            out_specs=[pl.BlockSpec((GATHER_WINDOW, 2 * EMB),
                                    index_map=lambda i: (i, 0))],
            core_axis_name=("core", "subcore"),
            dimension_semantics=(pltpu.PARALLEL,),
        )(idx_hbm, out_hbm)

    return gather_kernel(table_pairs, indices)


def _tile_copy(buf, out_hbm, sems, t, k):
    return pltpu.make_async_copy(
        buf, out_hbm.at[:, pl.ds(t * TILE, TILE)], sems.at[k])


def _mm_body(paired_ref, par_ref, w_ref, b_ref, out_hbm, lat_ref,
             b0, b1, b2, b3, tail_buf, sems, tail_sem):
    j = pl.program_id(0)
    bufs = (b0, b1, b2, b3)

    @pl.when(j == 0)
    def _():
        paired = paired_ref[...]
        par = par_ref[...]
        lat = jnp.where(par == 1, paired[:, EMB:], paired[:, :EMB])
        lat_ref[...] = lat.astype(jnp.bfloat16)

    acc = jax.lax.dot_general(
        lat_ref[...], w_ref[...].astype(jnp.bfloat16),
        dimension_numbers=(((1,), (1,)), ((), ())),
        preferred_element_type=jnp.float32,
    )
    res = (acc + b_ref[...]).astype(jnp.bfloat16)

    for k in range(NBUF):
        @pl.when((jax.lax.rem(j, NBUF) == k) & (j < NSTEPS - 1))
        def _(k=k):
            buf = bufs[k]
            @pl.when(j >= NBUF)
            def _():
                _tile_copy(buf, out_hbm, sems, j - NBUF, k).wait()
            buf[...] = res
            _tile_copy(buf, out_hbm, sems, j, k).start()

    @pl.when(j == NSTEPS - 1)
    def _():
        t_last = NSTEPS - 1
        tail_buf[...] = res[:, :TAIL]
        tail_copy = pltpu.make_async_copy(
            tail_buf, out_hbm.at[:, pl.ds(t_last * TILE, TAIL)], tail_sem)
        tail_copy.start()
        for t in range(t_last - NBUF, t_last):
            if t >= 0:
                _tile_copy(bufs[t % NBUF], out_hbm, sems, t, t % NBUF).wait()
        tail_copy.wait()


def _tc_matmul(paired, parity, W, b):
    b2d = b.reshape(1, VOCAB)
    return pl.pallas_call(
        _mm_body,
        grid=(NSTEPS,),
        in_specs=[
            pl.BlockSpec((BATCH, 2 * EMB), lambda j: (0, 0)),
            pl.BlockSpec((BATCH, 1), lambda j: (0, 0)),
            pl.BlockSpec((TILE, EMB), lambda j: (j, 0)),
            pl.BlockSpec((1, TILE), lambda j: (0, j)),
        ],
        out_specs=pl.BlockSpec(memory_space=pltpu.MemorySpace.HBM),
        out_shape=jax.ShapeDtypeStruct((BATCH, VOCAB), jnp.bfloat16),
        scratch_shapes=[
            pltpu.VMEM((BATCH, EMB), jnp.bfloat16),
            pltpu.VMEM((BATCH, TILE), jnp.bfloat16),
            pltpu.VMEM((BATCH, TILE), jnp.bfloat16),
            pltpu.VMEM((BATCH, TILE), jnp.bfloat16),
            pltpu.VMEM((BATCH, TILE), jnp.bfloat16),
            pltpu.VMEM((BATCH, TAIL), jnp.bfloat16),
            pltpu.SemaphoreType.DMA((NBUF,)),
            pltpu.SemaphoreType.DMA,
        ],
        compiler_params=pltpu.CompilerParams(
            dimension_semantics=("arbitrary",),
        ),
    )(paired, parity, W, b2d)


def kernel(context, emb_table, W, b):
    table_pairs = emb_table.reshape(VOCAB // 2, 2 * EMB)
    paired = _sc_gather_pairs(table_pairs, context // 2)
    parity = (context & 1).reshape(BATCH, 1)
    logits16 = _tc_matmul(paired, parity, W, b)
    return logits16.astype(jnp.float32)


# speedup vs baseline: 1.2329x; 1.0354x over previous
import jax
import jax.numpy as jnp
from jax.experimental import pallas as pl
from jax.experimental.pallas import tpu as pltpu
from jax.experimental.pallas import tpu_sc as plsc

VOCAB = 100000
EMB = 64
BATCH = 1024

GATHER_WINDOW = 128

TILE = 4096
NBUF = 4
NSTEPS = pl.cdiv(VOCAB, TILE)
TAIL = VOCAB - (NSTEPS - 1) * TILE


def _sc_gather_pairs(table_pairs, pair_idx):
    indices = pair_idx.reshape(1, BATCH)
    mesh = plsc.VectorSubcoreMesh(core_axis_name="core",
                                  subcore_axis_name="subcore")

    @pl.kernel(
        out_type=jax.ShapeDtypeStruct((BATCH, 2 * EMB), table_pairs.dtype),
        mesh=mesh,
    )
    def gather_kernel(table_hbm, idx_hbm, out_hbm):
        def body(idx_vmem, out_vmem):
            pltpu.sync_copy(table_hbm.at[idx_vmem.at[0]], out_vmem)

        pltpu.emit_pipeline(
            body,
            grid=(BATCH // GATHER_WINDOW,),
            in_specs=[pl.BlockSpec((1, GATHER_WINDOW),
                                   index_map=lambda i: (0, i))],
            out_specs=[pl.BlockSpec((GATHER_WINDOW, 2 * EMB),
                                    index_map=lambda i: (i, 0))],
            core_axis_name=("core", "subcore"),
            dimension_semantics=(pltpu.PARALLEL,),
        )(idx_hbm, out_hbm)

    return gather_kernel(table_pairs, indices)


def _tile_copy(buf, out_hbm, sems, t, k):
    return pltpu.make_async_copy(
        buf, out_hbm.at[:, pl.ds(t * TILE, TILE)], sems.at[k])


def _mm_body(paired_ref, par_ref, w_ref, b_ref, out_hbm, lat_ref,
             b0, b1, b2, b3, tail_buf, sems, tail_sem):
    j = pl.program_id(0)
    bufs = (b0, b1, b2, b3)

    @pl.when(j == 0)
    def _():
        paired = paired_ref[...]
        par = par_ref[...]
        lat = jnp.where(par == 1, paired[:, EMB:], paired[:, :EMB])
        lat_ref[...] = lat.astype(jnp.bfloat16)

    acc = jax.lax.dot_general(
        lat_ref[...], w_ref[...].astype(jnp.bfloat16),
        dimension_numbers=(((1,), (1,)), ((), ())),
        preferred_element_type=jnp.float32,
    )
    res = (acc + b_ref[...]).astype(jnp.bfloat16)

    for k in range(NBUF):
        @pl.when((jax.lax.rem(j, NBUF) == k) & (j < NSTEPS - 1))
        def _(k=k):
            buf = bufs[k]
            @pl.when(j >= NBUF)
            def _():
                _tile_copy(buf, out_hbm, sems, j - NBUF, k).wait()
            buf[...] = res
            _tile_copy(buf, out_hbm, sems, j, k).start()

    @pl.when(j == NSTEPS - 1)
    def _():
        t_last = NSTEPS - 1
        tail_buf[...] = res[:, :TAIL]
        tail_copy = pltpu.make_async_copy(
            tail_buf, out_hbm.at[:, pl.ds(t_last * TILE, TAIL)], tail_sem)
        tail_copy.start()
        for t in range(t_last - NBUF, t_last):
            if t >= 0:
                _tile_copy(bufs[t % NBUF], out_hbm, sems, t, t % NBUF).wait()
        tail_copy.wait()


def _tc_matmul(paired, parity, W, b):
    b2d = b.reshape(1, VOCAB)
    return pl.pallas_call(
        _mm_body,
        grid=(NSTEPS,),
        in_specs=[
            pl.BlockSpec((BATCH, 2 * EMB), lambda j: (0, 0)),
            pl.BlockSpec((BATCH, 1), lambda j: (0, 0)),
            pl.BlockSpec((TILE, EMB), lambda j: (j, 0)),
            pl.BlockSpec((1, TILE), lambda j: (0, j)),
        ],
        out_specs=pl.BlockSpec(memory_space=pltpu.MemorySpace.HBM),
        out_shape=jax.ShapeDtypeStruct((BATCH, VOCAB), jnp.bfloat16),
        scratch_shapes=[
            pltpu.VMEM((BATCH, EMB), jnp.bfloat16),
            pltpu.VMEM((BATCH, TILE), jnp.bfloat16),
            pltpu.VMEM((BATCH, TILE), jnp.bfloat16),
            pltpu.VMEM((BATCH, TILE), jnp.bfloat16),
            pltpu.VMEM((BATCH, TILE), jnp.bfloat16),
            pltpu.VMEM((BATCH, TAIL), jnp.bfloat16),
            pltpu.SemaphoreType.DMA((NBUF,)),
            pltpu.SemaphoreType.DMA,
        ],
        compiler_params=pltpu.CompilerParams(
            dimension_semantics=("arbitrary",),
        ),
    )(paired, parity, W, b2d)


def kernel(context, emb_table, W, b):
    table_pairs = emb_table.reshape(VOCAB // 2, 2 * EMB)
    paired = _sc_gather_pairs(table_pairs, context // 2)
    parity = (context & 1).reshape(BATCH, 1)
    logits16 = _tc_matmul(paired, parity, W, b)
    return logits16.astype(jnp.float32)
